# Initial kernel scaffold; baseline (speedup 1.0000x reference)
import jax
import jax.numpy as jnp
from jax.experimental import pallas as pl


def kernel(inputs, embedding):
    raise NotImplementedError("write your pallas kernel here")


# speedup vs baseline: 3.5374x; 3.5374x over previous
import jax
import jax.numpy as jnp
from jax import lax
from jax.experimental import pallas as pl
from jax.experimental.pallas import tpu as pltpu

K = 1024
D = 64
B = 16
HW = 1024
T = 512
C = HW // T
NTOK = B * HW
NSTEP = B * C
COMMIT = 0.25


def _vq_body(x_ref, emb_ref, enc_ref, q_ref, loss_ref, perp_ref,
             counts_ref, sse_ref):
    i = pl.program_id(0)

    @pl.when(i == 0)
    def _init():
        counts_ref[...] = jnp.zeros_like(counts_ref)
        sse_ref[0] = 0.0

    x = x_ref[0]
    emb = emb_ref[...]
    x2 = jnp.sum(x ** 2, axis=0)
    e2 = jnp.sum(emb ** 2, axis=1)
    xe = lax.dot_general(x, emb, (((0,), (1,)), ((), ())),
                         preferred_element_type=jnp.float32)
    scores = (x2[:, None] + e2[None, :]) - 2.0 * xe
    minval = jnp.min(scores, axis=1)
    iota_k = lax.broadcasted_iota(jnp.int32, (T, K), 1)
    idx = jnp.min(jnp.where(scores == minval[:, None], iota_k, K), axis=1)
    enc = (iota_k == idx[:, None]).astype(jnp.float32)
    enc_ref[...] = enc
    qT = lax.dot_general(emb, enc, (((0,), (1,)), ((), ())),
                         preferred_element_type=jnp.float32)
    q_ref[0] = qT
    diff = qT - x
    sse_ref[0] += jnp.sum(diff * diff)
    counts_ref[...] += jnp.sum(enc, axis=0)

    @pl.when(i == NSTEP - 1)
    def _fini():
        loss_ref[0, 0] = (1.0 + COMMIT) * sse_ref[0] / (NTOK * D)
        avg = counts_ref[...] * (1.0 / NTOK)
        perp_ref[0, 0] = jnp.exp(-jnp.sum(avg * jnp.log(avg + 1e-10)))


def kernel(inputs, embedding):
    xr = inputs.reshape(B, D, HW)
    enc, q, loss, perp = pl.pallas_call(
        _vq_body,
        grid=(NSTEP,),
        in_specs=[
            pl.BlockSpec((1, D, T), lambda i: (i // C, 0, i % C)),
            pl.BlockSpec((K, D), lambda i: (0, 0)),
        ],
        out_specs=[
            pl.BlockSpec((T, K), lambda i: (i, 0)),
            pl.BlockSpec((1, D, T), lambda i: (i // C, 0, i % C)),
            pl.BlockSpec((1, 1), lambda i: (0, 0), memory_space=pltpu.SMEM),
            pl.BlockSpec((1, 1), lambda i: (0, 0), memory_space=pltpu.SMEM),
        ],
        out_shape=[
            jax.ShapeDtypeStruct((NTOK, K), jnp.float32),
            jax.ShapeDtypeStruct((B, D, HW), jnp.float32),
            jax.ShapeDtypeStruct((1, 1), jnp.float32),
            jax.ShapeDtypeStruct((1, 1), jnp.float32),
        ],
        scratch_shapes=[
            pltpu.VMEM((K,), jnp.float32),
            pltpu.SMEM((1,), jnp.float32),
        ],
    )(xr, embedding)
    quantized = q.reshape(B, D, 32, 32)
    return (loss[0, 0], quantized, perp[0, 0], enc)


# speedup vs baseline: 3.9459x; 1.1155x over previous
import jax
import jax.numpy as jnp
from jax import lax
from jax.experimental import pallas as pl
from jax.experimental.pallas import tpu as pltpu

K = 1024
D = 64
B = 16
HW = 1024
T = 1024
C = HW // T
NTOK = B * HW
NSTEP = B * C
COMMIT = 0.25


def _vq_body(x_ref, emb_ref, enc_ref, q_ref, loss_ref, perp_ref,
             counts_ref, sse_ref):
    i = pl.program_id(0)

    @pl.when(i == 0)
    def _init():
        counts_ref[...] = jnp.zeros_like(counts_ref)
        sse_ref[0] = 0.0

    x = x_ref[0]
    emb = emb_ref[...]
    x2 = jnp.sum(x ** 2, axis=0)
    e2 = jnp.sum(emb ** 2, axis=1)
    xe = lax.dot_general(x, emb, (((0,), (1,)), ((), ())),
                         preferred_element_type=jnp.float32)
    scores = (x2[:, None] + e2[None, :]) - 2.0 * xe
    minval = jnp.min(scores, axis=1)
    iota_k = lax.broadcasted_iota(jnp.int32, (T, K), 1)
    idx = jnp.min(jnp.where(scores == minval[:, None], iota_k, K), axis=1)
    enc = (iota_k == idx[:, None]).astype(jnp.float32)
    enc_ref[...] = enc
    qT = lax.dot_general(emb, enc, (((0,), (1,)), ((), ())),
                         preferred_element_type=jnp.float32)
    q_ref[0] = qT
    diff = qT - x
    sse_ref[0] += jnp.sum(diff * diff)
    counts_ref[...] += jnp.sum(enc, axis=0)

    @pl.when(i == NSTEP - 1)
    def _fini():
        loss_ref[0, 0] = (1.0 + COMMIT) * sse_ref[0] / (NTOK * D)
        avg = counts_ref[...] * (1.0 / NTOK)
        perp_ref[0, 0] = jnp.exp(-jnp.sum(avg * jnp.log(avg + 1e-10)))


def kernel(inputs, embedding):
    xr = inputs.reshape(B, D, HW)
    enc, q, loss, perp = pl.pallas_call(
        _vq_body,
        grid=(NSTEP,),
        in_specs=[
            pl.BlockSpec((1, D, T), lambda i: (i // C, 0, i % C)),
            pl.BlockSpec((K, D), lambda i: (0, 0)),
        ],
        out_specs=[
            pl.BlockSpec((T, K), lambda i: (i, 0)),
            pl.BlockSpec((1, D, T), lambda i: (i // C, 0, i % C)),
            pl.BlockSpec((1, 1), lambda i: (0, 0), memory_space=pltpu.SMEM),
            pl.BlockSpec((1, 1), lambda i: (0, 0), memory_space=pltpu.SMEM),
        ],
        out_shape=[
            jax.ShapeDtypeStruct((NTOK, K), jnp.float32),
            jax.ShapeDtypeStruct((B, D, HW), jnp.float32),
            jax.ShapeDtypeStruct((1, 1), jnp.float32),
            jax.ShapeDtypeStruct((1, 1), jnp.float32),
        ],
        scratch_shapes=[
            pltpu.VMEM((K,), jnp.float32),
            pltpu.SMEM((1,), jnp.float32),
        ],
    )(xr, embedding)
    quantized = q.reshape(B, D, 32, 32)
    return (loss[0, 0], quantized, perp[0, 0], enc)
